# baseline (speedup 1.0000x reference)
import functools

import jax
import jax.numpy as jnp
from jax import lax
from jax.experimental import pallas as pl
from jax.experimental.pallas import tpu as pltpu
from jax.experimental.pallas import tpu_sc as plsc

N = 10000
E = 320000
LANES = 16
EDGE_COLS = 128
TILES = 16
ROWS_PER_TILE = 160
ROWS_TOTAL = TILES * ROWS_PER_TILE
E_PAD = ROWS_TOTAL * EDGE_COLS
RPT_FULL = ROWS_TOTAL // 32
NCHUNK = 624
ZCHUNK = 16
CR = 16
CRH = CR // 2
DH = 128

BN = 1000


@functools.cache
def _sc_pass(feature_split):
    mesh = plsc.VectorSubcoreMesh(core_axis_name="c", subcore_axis_name="s")
    rpt = ROWS_PER_TILE if feature_split else RPT_FULL

    @functools.partial(
        pl.kernel,
        mesh=mesh,
        out_type=jax.ShapeDtypeStruct((2 * N, DH), jnp.float32),
        scratch_types=[
            pltpu.VMEM((CR, EDGE_COLS), jnp.int32),
            pltpu.VMEM((CR, EDGE_COLS), jnp.int32),
            pltpu.VMEM((CR, EDGE_COLS), jnp.float32),
            pltpu.VMEM((2, EDGE_COLS, DH), jnp.float32),
            pltpu.VMEM((ZCHUNK, DH), jnp.float32),
            pltpu.VMEM_SHARED((N, DH), jnp.float32),
            pltpu.SemaphoreType.DMA,
            pltpu.SemaphoreType.DMA,
            pltpu.SemaphoreType.DMA,
            pltpu.SemaphoreType.DMA,
        ],
    )
    def agg(ys_hbm, src_hbm, dst_hbm, ew_hbm, out_hbm,
            src_v, dst_v, ew_v, rows_v, zero_v, acc_sh,
            sem0, sem1, tsem0, tsem1):
        c = lax.axis_index("c")
        s = lax.axis_index("s")

        zf = jnp.zeros((LANES,), jnp.float32)

        def zrow(r, carry):
            for k in range(DH // LANES):
                zero_v[r, pl.ds(k * LANES, LANES)] = zf
            return carry

        lax.fori_loop(0, ZCHUNK, zrow, 0)
        n0 = s * NCHUNK
        nz = jnp.where(s == TILES - 1, (N - (TILES - 1) * NCHUNK) // ZCHUNK,
                       NCHUNK // ZCHUNK)

        def zcopy(t, carry):
            pltpu.sync_copy(zero_v, acc_sh.at[pl.ds(n0 + t * ZCHUNK, ZCHUNK)])
            return carry

        lax.fori_loop(0, nz, zcopy, 0)
        plsc.subcore_barrier()

        if feature_split:
            row0 = s * ROWS_PER_TILE
        else:
            row0 = (c * TILES + s) * RPT_FULL

        gsems = (sem0, sem1)
        tsems = (tsem0, tsem1)

        def gather_start(b, j):
            pltpu.async_copy(ys_hbm.at[src_v.at[j]], rows_v.at[b], gsems[b])

        def gather_wait(b, j):
            pltpu.make_async_copy(ys_hbm.at[src_v.at[j]], rows_v.at[b],
                                  gsems[b]).wait()

        def scatter_start(b, j):
            pltpu.async_copy(rows_v.at[b], acc_sh.at[dst_v.at[j]], tsems[b],
                             add=True)

        def scatter_wait(b):
            pltpu.make_async_copy(rows_v.at[b], acc_sh.at[dst_v.at[0]],
                                  tsems[b]).wait()

        def scale(b, j):
            @plsc.parallel_loop(0, EDGE_COLS // LANES, unroll=2)
            def _(g):
                wv = ew_v[j, pl.ds(g * LANES, LANES)]
                e0 = g * LANES
                for i in range(LANES):
                    w = wv[i]
                    for k in range(DH // LANES):
                        rows_v[b, e0 + i, pl.ds(k * LANES, LANES)] = (
                            rows_v[b, e0 + i, pl.ds(k * LANES, LANES)] * w)

        def chunk(ci, carry):
            r0 = row0 + ci * CR
            if feature_split:
                pltpu.sync_copy(src_hbm.at[c, pl.ds(r0, CR)], src_v)
            else:
                pltpu.sync_copy(src_hbm.at[pl.ds(r0, CR)], src_v)
            pltpu.sync_copy(dst_hbm.at[pl.ds(r0, CR)], dst_v)
            pltpu.sync_copy(ew_hbm.at[pl.ds(r0, CR)], ew_v)

            gather_start(0, 0)

            def pair(p, c2):
                j0 = p * 2

                @pl.when(p > 0)
                def _():
                    scatter_wait(1)

                gather_start(1, j0 + 1)
                gather_wait(0, j0)
                scale(0, j0)
                scatter_start(0, j0)

                @pl.when(p < CRH - 1)
                def _():
                    scatter_wait(0)
                    gather_start(0, j0 + 2)

                gather_wait(1, j0 + 1)
                scale(1, j0 + 1)
                scatter_start(1, j0 + 1)
                return c2

            lax.fori_loop(0, CRH, pair, 0)
            scatter_wait(0)
            scatter_wait(1)
            return carry

        lax.fori_loop(0, rpt // CR, chunk, 0)
        plsc.subcore_barrier()

        last = N - (TILES - 1) * NCHUNK

        @pl.when(s < TILES - 1)
        def _():
            pltpu.sync_copy(acc_sh.at[pl.ds(n0, NCHUNK)],
                            out_hbm.at[pl.ds(c * N + n0, NCHUNK)])

        @pl.when(s == TILES - 1)
        def _():
            pltpu.sync_copy(acc_sh.at[pl.ds(n0, last)],
                            out_hbm.at[pl.ds(c * N + n0, last)])

    return agg


def _tc_first(x, w1, degp):
    def body(x_ref, w_ref, deg_ref, ys_ref, dis_ref):
        deg = deg_ref[0, :, 0:1] + deg_ref[1, :, 0:1] + 1.0
        dis = lax.rsqrt(deg)
        xw = jnp.dot(x_ref[...], w_ref[...],
                     preferred_element_type=jnp.float32)
        ys = xw * dis
        ys_ref[0] = ys[:, :128]
        ys_ref[1] = ys[:, 128:]
        dis_ref[...] = dis

    return pl.pallas_call(
        body,
        grid=(N // BN,),
        in_specs=[
            pl.BlockSpec((BN, 128), lambda i: (i, 0)),
            pl.BlockSpec((128, 256), lambda i: (0, 0)),
            pl.BlockSpec((2, BN, 128), lambda i: (0, i, 0)),
        ],
        out_specs=[
            pl.BlockSpec((2, BN, 128), lambda i: (0, i, 0)),
            pl.BlockSpec((BN, 1), lambda i: (i, 0)),
        ],
        out_shape=[
            jax.ShapeDtypeStruct((2, N, 128), jnp.float32),
            jax.ShapeDtypeStruct((N, 1), jnp.float32),
        ],
    )(x, w1, degp)


def _tc_mid(agg, ys, dis2, b2d, w, d_in_h, d_out, split_out):
    doh = d_out // 2

    def body(a_ref, ys_ref, dis_ref, b_ref, w_ref, out_ref):
        dis = dis_ref[...]
        h0 = jnp.maximum((a_ref[0] + ys_ref[0]) * dis + b_ref[0], 0.0)
        h1 = jnp.maximum((a_ref[1] + ys_ref[1]) * dis + b_ref[1], 0.0)
        out = jnp.dot(h0, w_ref[:d_in_h, :],
                      preferred_element_type=jnp.float32)
        out = out + jnp.dot(h1, w_ref[d_in_h:, :],
                            preferred_element_type=jnp.float32)
        ysn = out * dis
        if split_out:
            out_ref[0] = ysn[:, :doh]
            out_ref[1] = ysn[:, doh:]
        else:
            out_ref[...] = ysn

    if split_out:
        out_spec = pl.BlockSpec((2, BN, doh), lambda i: (0, i, 0))
        out_shape = jax.ShapeDtypeStruct((2, N, doh), jnp.float32)
    else:
        out_spec = pl.BlockSpec((BN, d_out), lambda i: (i, 0))
        out_shape = jax.ShapeDtypeStruct((N, d_out), jnp.float32)

    return pl.pallas_call(
        body,
        grid=(N // BN,),
        in_specs=[
            pl.BlockSpec((2, BN, d_in_h), lambda i: (0, i, 0)),
            pl.BlockSpec((2, BN, d_in_h), lambda i: (0, i, 0)),
            pl.BlockSpec((BN, 1), lambda i: (i, 0)),
            pl.BlockSpec((2, 1, d_in_h), lambda i: (0, 0, 0)),
            pl.BlockSpec((2 * d_in_h, d_out), lambda i: (0, 0)),
        ],
        out_specs=out_spec,
        out_shape=out_shape,
    )(agg, ys, dis2, b2d, w)


def _tc_final(aggp, ys, dis2, b2d):
    def body(a_ref, ys_ref, dis_ref, b_ref, out_ref):
        dis = dis_ref[...]
        a = a_ref[0] + a_ref[1]
        out_ref[...] = jnp.maximum((a + ys_ref[...]) * dis + b_ref[...], 0.0)

    return pl.pallas_call(
        body,
        grid=(N // BN,),
        in_specs=[
            pl.BlockSpec((2, BN, 128), lambda i: (0, i, 0)),
            pl.BlockSpec((BN, 128), lambda i: (i, 0)),
            pl.BlockSpec((BN, 1), lambda i: (i, 0)),
            pl.BlockSpec((1, 128), lambda i: (0, 0)),
        ],
        out_specs=pl.BlockSpec((BN, 128), lambda i: (i, 0)),
        out_shape=jax.ShapeDtypeStruct((N, 128), jnp.float32),
    )(aggp, ys, dis2, b2d)


def kernel(x, edge_index, edge_features, W1, b1, Wh, bh, W2, b2):
    src = edge_index[0].astype(jnp.int32)
    dst = edge_index[1].astype(jnp.int32)
    ew = edge_features.astype(jnp.float32)

    pad = E_PAD - E
    src_p = jnp.concatenate([src, jnp.zeros((pad,), jnp.int32)])
    dst_p = jnp.concatenate([dst, jnp.zeros((pad,), jnp.int32)])
    ew_p = jnp.concatenate([ew, jnp.zeros((pad,), jnp.float32)])
    src2 = jnp.stack([src_p, src_p + N]).reshape(2, ROWS_TOTAL, EDGE_COLS)
    dstr = dst_p.reshape(ROWS_TOTAL, EDGE_COLS)
    ewr = ew_p.reshape(ROWS_TOTAL, EDGE_COLS)

    ones128 = jnp.ones((N, 128), jnp.float32)
    degp = _sc_pass(False)(ones128, src2[0], dstr, ewr).reshape(2, N, 128)

    b1_2d = b1.reshape(2, 1, 128)
    bh_2d = bh.reshape(2, 1, 128)
    b2_2d = b2.reshape(1, 128)

    ys1, dis2 = _tc_first(x, W1, degp)
    a1 = _sc_pass(True)(ys1.reshape(2 * N, 128), src2, dstr, ewr)
    ys2 = _tc_mid(a1.reshape(2, N, 128), ys1, dis2, b1_2d, Wh, 128, 256,
                  split_out=True)
    a2 = _sc_pass(True)(ys2.reshape(2 * N, 128), src2, dstr, ewr)
    ys3 = _tc_mid(a2.reshape(2, N, 128), ys2, dis2, bh_2d, W2, 128, 128,
                  split_out=False)
    a3p = _sc_pass(False)(ys3, src2[0], dstr, ewr).reshape(2, N, 128)
    return _tc_final(a3p, ys3, dis2, b2_2d)


# speedup vs baseline: 1.0017x; 1.0017x over previous
import functools

import jax
import jax.numpy as jnp
from jax import lax
from jax.experimental import pallas as pl
from jax.experimental.pallas import tpu as pltpu
from jax.experimental.pallas import tpu_sc as plsc

N = 10000
E = 320000
LANES = 16
EDGE_COLS = 128
TILES = 16
ROWS_PER_TILE = 160
ROWS_TOTAL = TILES * ROWS_PER_TILE
E_PAD = ROWS_TOTAL * EDGE_COLS
RPT_FULL = ROWS_TOTAL // 32
NCHUNK = 624
ZCHUNK = 16
CR = 16
CRH = CR // 2
DH = 128
GS = 4

BN = 1000


@functools.cache
def _sc_pass(feature_split):
    mesh = plsc.VectorSubcoreMesh(core_axis_name="c", subcore_axis_name="s")
    rpt = ROWS_PER_TILE if feature_split else RPT_FULL

    @functools.partial(
        pl.kernel,
        mesh=mesh,
        out_type=jax.ShapeDtypeStruct((2 * N, DH), jnp.float32),
        scratch_types=[
            pltpu.VMEM((CR, EDGE_COLS), jnp.int32),
            pltpu.VMEM((CR, EDGE_COLS), jnp.int32),
            pltpu.VMEM((CR, EDGE_COLS), jnp.float32),
            pltpu.VMEM((2, EDGE_COLS, DH), jnp.float32),
            pltpu.VMEM((ZCHUNK, DH), jnp.float32),
            pltpu.VMEM_SHARED((N, DH), jnp.float32),
            pltpu.SemaphoreType.DMA,
            pltpu.SemaphoreType.DMA,
            pltpu.SemaphoreType.DMA,
            pltpu.SemaphoreType.DMA,
        ],
    )
    def agg(ys_hbm, src_hbm, dst_hbm, ew_hbm, out_hbm,
            src_v, dst_v, ew_v, rows_v, zero_v, acc_sh,
            sem0, sem1, tsem0, tsem1):
        c = lax.axis_index("c")
        s = lax.axis_index("s")

        zf = jnp.zeros((LANES,), jnp.float32)

        def zrow(r, carry):
            for k in range(DH // LANES):
                zero_v[r, pl.ds(k * LANES, LANES)] = zf
            return carry

        lax.fori_loop(0, ZCHUNK, zrow, 0)
        n0 = s * NCHUNK
        nz = jnp.where(s == TILES - 1, (N - (TILES - 1) * NCHUNK) // ZCHUNK,
                       NCHUNK // ZCHUNK)

        def zcopy(t, carry):
            pltpu.sync_copy(zero_v, acc_sh.at[pl.ds(n0 + t * ZCHUNK, ZCHUNK)])
            return carry

        lax.fori_loop(0, nz, zcopy, 0)
        plsc.subcore_barrier()

        if feature_split:
            row0 = s * ROWS_PER_TILE
        else:
            row0 = (c * TILES + s) * RPT_FULL

        gsems = (sem0, sem1)
        tsems = (tsem0, tsem1)

        def gather_start(b, j):
            for q in range(GS):
                pltpu.async_copy(
                    ys_hbm.at[src_v.at[j, pl.ds(q * (EDGE_COLS // GS),
                                                EDGE_COLS // GS)]],
                    rows_v.at[b, pl.ds(q * (EDGE_COLS // GS),
                                       EDGE_COLS // GS)],
                    gsems[b])

        def gather_wait(b, j):
            pltpu.make_async_copy(ys_hbm.at[src_v.at[j]], rows_v.at[b],
                                  gsems[b]).wait()

        def scatter_start(b, j):
            pltpu.async_copy(rows_v.at[b], acc_sh.at[dst_v.at[j]], tsems[b],
                             add=True)

        def scatter_wait(b):
            pltpu.make_async_copy(rows_v.at[b], acc_sh.at[dst_v.at[0]],
                                  tsems[b]).wait()

        def scale(b, j):
            @plsc.parallel_loop(0, EDGE_COLS // LANES, unroll=2)
            def _(g):
                wv = ew_v[j, pl.ds(g * LANES, LANES)]
                e0 = g * LANES
                for i in range(LANES):
                    w = wv[i]
                    for k in range(DH // LANES):
                        rows_v[b, e0 + i, pl.ds(k * LANES, LANES)] = (
                            rows_v[b, e0 + i, pl.ds(k * LANES, LANES)] * w)

        def chunk(ci, carry):
            r0 = row0 + ci * CR
            if feature_split:
                pltpu.sync_copy(src_hbm.at[c, pl.ds(r0, CR)], src_v)
            else:
                pltpu.sync_copy(src_hbm.at[pl.ds(r0, CR)], src_v)
            pltpu.sync_copy(dst_hbm.at[pl.ds(r0, CR)], dst_v)
            pltpu.sync_copy(ew_hbm.at[pl.ds(r0, CR)], ew_v)

            gather_start(0, 0)

            def pair(p, c2):
                j0 = p * 2

                @pl.when(p > 0)
                def _():
                    scatter_wait(1)

                gather_start(1, j0 + 1)
                gather_wait(0, j0)
                scale(0, j0)
                scatter_start(0, j0)

                @pl.when(p < CRH - 1)
                def _():
                    scatter_wait(0)
                    gather_start(0, j0 + 2)

                gather_wait(1, j0 + 1)
                scale(1, j0 + 1)
                scatter_start(1, j0 + 1)
                return c2

            lax.fori_loop(0, CRH, pair, 0)
            scatter_wait(0)
            scatter_wait(1)
            return carry

        lax.fori_loop(0, rpt // CR, chunk, 0)
        plsc.subcore_barrier()

        last = N - (TILES - 1) * NCHUNK

        @pl.when(s < TILES - 1)
        def _():
            pltpu.sync_copy(acc_sh.at[pl.ds(n0, NCHUNK)],
                            out_hbm.at[pl.ds(c * N + n0, NCHUNK)])

        @pl.when(s == TILES - 1)
        def _():
            pltpu.sync_copy(acc_sh.at[pl.ds(n0, last)],
                            out_hbm.at[pl.ds(c * N + n0, last)])

    return agg


def _tc_first(x, w1, degp):
    def body(x_ref, w_ref, deg_ref, ys_ref, dis_ref):
        deg = deg_ref[0, :, 0:1] + deg_ref[1, :, 0:1] + 1.0
        dis = lax.rsqrt(deg)
        xw = jnp.dot(x_ref[...], w_ref[...],
                     preferred_element_type=jnp.float32)
        ys = xw * dis
        ys_ref[0] = ys[:, :128]
        ys_ref[1] = ys[:, 128:]
        dis_ref[...] = dis

    return pl.pallas_call(
        body,
        grid=(N // BN,),
        in_specs=[
            pl.BlockSpec((BN, 128), lambda i: (i, 0)),
            pl.BlockSpec((128, 256), lambda i: (0, 0)),
            pl.BlockSpec((2, BN, 128), lambda i: (0, i, 0)),
        ],
        out_specs=[
            pl.BlockSpec((2, BN, 128), lambda i: (0, i, 0)),
            pl.BlockSpec((BN, 1), lambda i: (i, 0)),
        ],
        out_shape=[
            jax.ShapeDtypeStruct((2, N, 128), jnp.float32),
            jax.ShapeDtypeStruct((N, 1), jnp.float32),
        ],
    )(x, w1, degp)


def _tc_mid(agg, ys, dis2, b2d, w, d_in_h, d_out, split_out):
    doh = d_out // 2

    def body(a_ref, ys_ref, dis_ref, b_ref, w_ref, out_ref):
        dis = dis_ref[...]
        h0 = jnp.maximum((a_ref[0] + ys_ref[0]) * dis + b_ref[0], 0.0)
        h1 = jnp.maximum((a_ref[1] + ys_ref[1]) * dis + b_ref[1], 0.0)
        out = jnp.dot(h0, w_ref[:d_in_h, :],
                      preferred_element_type=jnp.float32)
        out = out + jnp.dot(h1, w_ref[d_in_h:, :],
                            preferred_element_type=jnp.float32)
        ysn = out * dis
        if split_out:
            out_ref[0] = ysn[:, :doh]
            out_ref[1] = ysn[:, doh:]
        else:
            out_ref[...] = ysn

    if split_out:
        out_spec = pl.BlockSpec((2, BN, doh), lambda i: (0, i, 0))
        out_shape = jax.ShapeDtypeStruct((2, N, doh), jnp.float32)
    else:
        out_spec = pl.BlockSpec((BN, d_out), lambda i: (i, 0))
        out_shape = jax.ShapeDtypeStruct((N, d_out), jnp.float32)

    return pl.pallas_call(
        body,
        grid=(N // BN,),
        in_specs=[
            pl.BlockSpec((2, BN, d_in_h), lambda i: (0, i, 0)),
            pl.BlockSpec((2, BN, d_in_h), lambda i: (0, i, 0)),
            pl.BlockSpec((BN, 1), lambda i: (i, 0)),
            pl.BlockSpec((2, 1, d_in_h), lambda i: (0, 0, 0)),
            pl.BlockSpec((2 * d_in_h, d_out), lambda i: (0, 0)),
        ],
        out_specs=out_spec,
        out_shape=out_shape,
    )(agg, ys, dis2, b2d, w)


def _tc_final(aggp, ys, dis2, b2d):
    def body(a_ref, ys_ref, dis_ref, b_ref, out_ref):
        dis = dis_ref[...]
        a = a_ref[0] + a_ref[1]
        out_ref[...] = jnp.maximum((a + ys_ref[...]) * dis + b_ref[...], 0.0)

    return pl.pallas_call(
        body,
        grid=(N // BN,),
        in_specs=[
            pl.BlockSpec((2, BN, 128), lambda i: (0, i, 0)),
            pl.BlockSpec((BN, 128), lambda i: (i, 0)),
            pl.BlockSpec((BN, 1), lambda i: (i, 0)),
            pl.BlockSpec((1, 128), lambda i: (0, 0)),
        ],
        out_specs=pl.BlockSpec((BN, 128), lambda i: (i, 0)),
        out_shape=jax.ShapeDtypeStruct((N, 128), jnp.float32),
    )(aggp, ys, dis2, b2d)


def kernel(x, edge_index, edge_features, W1, b1, Wh, bh, W2, b2):
    src = edge_index[0].astype(jnp.int32)
    dst = edge_index[1].astype(jnp.int32)
    ew = edge_features.astype(jnp.float32)

    pad = E_PAD - E
    src_p = jnp.concatenate([src, jnp.zeros((pad,), jnp.int32)])
    dst_p = jnp.concatenate([dst, jnp.zeros((pad,), jnp.int32)])
    ew_p = jnp.concatenate([ew, jnp.zeros((pad,), jnp.float32)])
    src2 = jnp.stack([src_p, src_p + N]).reshape(2, ROWS_TOTAL, EDGE_COLS)
    dstr = dst_p.reshape(ROWS_TOTAL, EDGE_COLS)
    ewr = ew_p.reshape(ROWS_TOTAL, EDGE_COLS)

    ones128 = jnp.ones((N, 128), jnp.float32)
    degp = _sc_pass(False)(ones128, src2[0], dstr, ewr).reshape(2, N, 128)

    b1_2d = b1.reshape(2, 1, 128)
    bh_2d = bh.reshape(2, 1, 128)
    b2_2d = b2.reshape(1, 128)

    ys1, dis2 = _tc_first(x, W1, degp)
    a1 = _sc_pass(True)(ys1.reshape(2 * N, 128), src2, dstr, ewr)
    ys2 = _tc_mid(a1.reshape(2, N, 128), ys1, dis2, b1_2d, Wh, 128, 256,
                  split_out=True)
    a2 = _sc_pass(True)(ys2.reshape(2 * N, 128), src2, dstr, ewr)
    ys3 = _tc_mid(a2.reshape(2, N, 128), ys2, dis2, bh_2d, W2, 128, 128,
                  split_out=False)
    a3p = _sc_pass(False)(ys3, src2[0], dstr, ewr).reshape(2, N, 128)
    return _tc_final(a3p, ys3, dis2, b2_2d)


# speedup vs baseline: 1.1700x; 1.1680x over previous
import functools

import jax
import jax.numpy as jnp
from jax import lax
from jax.experimental import pallas as pl
from jax.experimental.pallas import tpu as pltpu
from jax.experimental.pallas import tpu_sc as plsc

N = 10000
E = 320000
LANES = 16
EDGE_COLS = 128
TILES = 16
ROWS_PER_TILE = 160
ROWS_TOTAL = TILES * ROWS_PER_TILE
E_PAD = ROWS_TOTAL * EDGE_COLS
RPT_FULL = ROWS_TOTAL // 32
NCHUNK = 624
ZCHUNK = 16
CR = 16
CRH = CR // 2
DH = 128
GS = 4

BN = 1000


@functools.cache
def _sc_pass(feature_split):
    mesh = plsc.VectorSubcoreMesh(core_axis_name="c", subcore_axis_name="s")
    rpt = ROWS_PER_TILE if feature_split else RPT_FULL

    @functools.partial(
        pl.kernel,
        mesh=mesh,
        out_type=jax.ShapeDtypeStruct((2 * N, DH), jnp.float32),
        scratch_types=[
            pltpu.VMEM((CR, EDGE_COLS), jnp.int32),
            pltpu.VMEM((CR, EDGE_COLS), jnp.int32),
            pltpu.VMEM((CR, EDGE_COLS), jnp.float32),
            pltpu.VMEM((2, EDGE_COLS, DH), jnp.float32),
            pltpu.VMEM((ZCHUNK, DH), jnp.float32),
            pltpu.VMEM_SHARED((N, DH), jnp.float32),
            pltpu.SemaphoreType.DMA,
            pltpu.SemaphoreType.DMA,
            pltpu.SemaphoreType.DMA,
            pltpu.SemaphoreType.DMA,
        ],
    )
    def agg(ys_hbm, src_hbm, dst_hbm, ew_hbm, out_hbm,
            src_v, dst_v, ew_v, rows_v, zero_v, acc_sh,
            sem0, sem1, tsem0, tsem1):
        c = lax.axis_index("c")
        s = lax.axis_index("s")

        zf = jnp.zeros((LANES,), jnp.float32)

        def zrow(r, carry):
            for k in range(DH // LANES):
                zero_v[r, pl.ds(k * LANES, LANES)] = zf
            return carry

        lax.fori_loop(0, ZCHUNK, zrow, 0)
        n0 = s * NCHUNK
        nz = jnp.where(s == TILES - 1, (N - (TILES - 1) * NCHUNK) // ZCHUNK,
                       NCHUNK // ZCHUNK)

        def zcopy(t, carry):
            pltpu.sync_copy(zero_v, acc_sh.at[pl.ds(n0 + t * ZCHUNK, ZCHUNK)])
            return carry

        lax.fori_loop(0, nz, zcopy, 0)
        plsc.subcore_barrier()

        if feature_split:
            row0 = s * ROWS_PER_TILE
        else:
            row0 = (c * TILES + s) * RPT_FULL

        gsems = (sem0, sem1)
        tsems = (tsem0, tsem1)

        def gather_start(b, j):
            for q in range(GS):
                pltpu.async_copy(
                    ys_hbm.at[src_v.at[j, pl.ds(q * (EDGE_COLS // GS),
                                                EDGE_COLS // GS)]],
                    rows_v.at[b, pl.ds(q * (EDGE_COLS // GS),
                                       EDGE_COLS // GS)],
                    gsems[b])

        def gather_wait(b, j):
            pltpu.make_async_copy(ys_hbm.at[src_v.at[j]], rows_v.at[b],
                                  gsems[b]).wait()

        def scatter_start(b, j):
            pltpu.async_copy(rows_v.at[b], acc_sh.at[dst_v.at[j]], tsems[b],
                             add=True)

        def scatter_wait(b):
            pltpu.make_async_copy(rows_v.at[b], acc_sh.at[dst_v.at[0]],
                                  tsems[b]).wait()

        def scale(b, j):
            @plsc.parallel_loop(0, EDGE_COLS // LANES, unroll=2)
            def _(g):
                wv = ew_v[j, pl.ds(g * LANES, LANES)]
                e0 = g * LANES
                for i in range(LANES):
                    w = wv[i]
                    for k in range(DH // LANES):
                        rows_v[b, e0 + i, pl.ds(k * LANES, LANES)] = (
                            rows_v[b, e0 + i, pl.ds(k * LANES, LANES)] * w)

        def chunk(ci, carry):
            r0 = row0 + ci * CR
            if feature_split:
                pltpu.sync_copy(src_hbm.at[c, pl.ds(r0, CR)], src_v)
            else:
                pltpu.sync_copy(src_hbm.at[pl.ds(r0, CR)], src_v)
            pltpu.sync_copy(dst_hbm.at[pl.ds(r0, CR)], dst_v)
            pltpu.sync_copy(ew_hbm.at[pl.ds(r0, CR)], ew_v)

            gather_start(0, 0)

            def pair(p, c2):
                j0 = p * 2

                @pl.when(p > 0)
                def _():
                    scatter_wait(1)

                gather_start(1, j0 + 1)
                gather_wait(0, j0)
                scale(0, j0)
                scatter_start(0, j0)

                @pl.when(p < CRH - 1)
                def _():
                    scatter_wait(0)
                    gather_start(0, j0 + 2)

                gather_wait(1, j0 + 1)
                scale(1, j0 + 1)
                scatter_start(1, j0 + 1)
                return c2

            lax.fori_loop(0, CRH, pair, 0)
            scatter_wait(0)
            scatter_wait(1)
            return carry

        lax.fori_loop(0, rpt // CR, chunk, 0)
        plsc.subcore_barrier()

        last = N - (TILES - 1) * NCHUNK

        @pl.when(s < TILES - 1)
        def _():
            pltpu.sync_copy(acc_sh.at[pl.ds(n0, NCHUNK)],
                            out_hbm.at[pl.ds(c * N + n0, NCHUNK)])

        @pl.when(s == TILES - 1)
        def _():
            pltpu.sync_copy(acc_sh.at[pl.ds(n0, last)],
                            out_hbm.at[pl.ds(c * N + n0, last)])

    return agg


@functools.cache
def _sc_deg():
    mesh = plsc.VectorSubcoreMesh(core_axis_name="c", subcore_axis_name="s")

    @functools.partial(
        pl.kernel,
        mesh=mesh,
        out_type=jax.ShapeDtypeStruct((2 * N, DH), jnp.float32),
        scratch_types=[
            pltpu.VMEM((CR, EDGE_COLS), jnp.int32),
            pltpu.VMEM((CR, EDGE_COLS), jnp.float32),
            pltpu.VMEM((2, EDGE_COLS, DH), jnp.float32),
            pltpu.VMEM((ZCHUNK, DH), jnp.float32),
            pltpu.VMEM_SHARED((N, DH), jnp.float32),
            pltpu.SemaphoreType.DMA,
            pltpu.SemaphoreType.DMA,
        ],
    )
    def deg(dst_hbm, ew_hbm, out_hbm,
            dst_v, ew_v, rows_v, zero_v, acc_sh, tsem0, tsem1):
        c = lax.axis_index("c")
        s = lax.axis_index("s")

        zf = jnp.zeros((LANES,), jnp.float32)

        def zrow(r, carry):
            for k in range(DH // LANES):
                zero_v[r, pl.ds(k * LANES, LANES)] = zf
            return carry

        lax.fori_loop(0, ZCHUNK, zrow, 0)
        n0 = s * NCHUNK
        nz = jnp.where(s == TILES - 1, (N - (TILES - 1) * NCHUNK) // ZCHUNK,
                       NCHUNK // ZCHUNK)

        def zcopy(t, carry):
            pltpu.sync_copy(zero_v, acc_sh.at[pl.ds(n0 + t * ZCHUNK, ZCHUNK)])
            return carry

        lax.fori_loop(0, nz, zcopy, 0)
        plsc.subcore_barrier()

        row0 = (c * TILES + s) * RPT_FULL
        tsems = (tsem0, tsem1)

        def build(b, j):
            @plsc.parallel_loop(0, EDGE_COLS // LANES, unroll=2)
            def _(g):
                wv = ew_v[j, pl.ds(g * LANES, LANES)]
                e0 = g * LANES
                for i in range(LANES):
                    wb = jnp.full((LANES,), wv[i], jnp.float32)
                    for k in range(DH // LANES):
                        rows_v[b, e0 + i, pl.ds(k * LANES, LANES)] = wb

        def scatter_start(b, j):
            pltpu.async_copy(rows_v.at[b], acc_sh.at[dst_v.at[j]], tsems[b],
                             add=True)

        def scatter_wait(b):
            pltpu.make_async_copy(rows_v.at[b], acc_sh.at[dst_v.at[0]],
                                  tsems[b]).wait()

        def chunk(ci, carry):
            r0 = row0 + ci * CR
            pltpu.sync_copy(dst_hbm.at[pl.ds(r0, CR)], dst_v)
            pltpu.sync_copy(ew_hbm.at[pl.ds(r0, CR)], ew_v)

            def pair(p, c2):
                j0 = p * 2

                @pl.when(p > 0)
                def _():
                    scatter_wait(0)
                    scatter_wait(1)

                build(0, j0)
                scatter_start(0, j0)
                build(1, j0 + 1)
                scatter_start(1, j0 + 1)
                return c2

            lax.fori_loop(0, CRH, pair, 0)
            scatter_wait(0)
            scatter_wait(1)
            return carry

        lax.fori_loop(0, RPT_FULL // CR, chunk, 0)
        plsc.subcore_barrier()

        last = N - (TILES - 1) * NCHUNK

        @pl.when(s < TILES - 1)
        def _():
            pltpu.sync_copy(acc_sh.at[pl.ds(n0, NCHUNK)],
                            out_hbm.at[pl.ds(c * N + n0, NCHUNK)])

        @pl.when(s == TILES - 1)
        def _():
            pltpu.sync_copy(acc_sh.at[pl.ds(n0, last)],
                            out_hbm.at[pl.ds(c * N + n0, last)])

    return deg


def _tc_first(x, w1, degp):
    def body(x_ref, w_ref, deg_ref, ys_ref, dis_ref):
        deg = deg_ref[0, :, 0:1] + deg_ref[1, :, 0:1] + 1.0
        dis = lax.rsqrt(deg)
        xw = jnp.dot(x_ref[...], w_ref[...],
                     preferred_element_type=jnp.float32)
        ys = xw * dis
        ys_ref[0] = ys[:, :128]
        ys_ref[1] = ys[:, 128:]
        dis_ref[...] = dis

    return pl.pallas_call(
        body,
        grid=(N // BN,),
        in_specs=[
            pl.BlockSpec((BN, 128), lambda i: (i, 0)),
            pl.BlockSpec((128, 256), lambda i: (0, 0)),
            pl.BlockSpec((2, BN, 128), lambda i: (0, i, 0)),
        ],
        out_specs=[
            pl.BlockSpec((2, BN, 128), lambda i: (0, i, 0)),
            pl.BlockSpec((BN, 1), lambda i: (i, 0)),
        ],
        out_shape=[
            jax.ShapeDtypeStruct((2, N, 128), jnp.float32),
            jax.ShapeDtypeStruct((N, 1), jnp.float32),
        ],
    )(x, w1, degp)


def _tc_mid(agg, ys, dis2, b2d, w, d_in_h, d_out, split_out):
    doh = d_out // 2

    def body(a_ref, ys_ref, dis_ref, b_ref, w_ref, out_ref):
        dis = dis_ref[...]
        h0 = jnp.maximum((a_ref[0] + ys_ref[0]) * dis + b_ref[0], 0.0)
        h1 = jnp.maximum((a_ref[1] + ys_ref[1]) * dis + b_ref[1], 0.0)
        out = jnp.dot(h0, w_ref[:d_in_h, :],
                      preferred_element_type=jnp.float32)
        out = out + jnp.dot(h1, w_ref[d_in_h:, :],
                            preferred_element_type=jnp.float32)
        ysn = out * dis
        if split_out:
            out_ref[0] = ysn[:, :doh]
            out_ref[1] = ysn[:, doh:]
        else:
            out_ref[...] = ysn

    if split_out:
        out_spec = pl.BlockSpec((2, BN, doh), lambda i: (0, i, 0))
        out_shape = jax.ShapeDtypeStruct((2, N, doh), jnp.float32)
    else:
        out_spec = pl.BlockSpec((BN, d_out), lambda i: (i, 0))
        out_shape = jax.ShapeDtypeStruct((N, d_out), jnp.float32)

    return pl.pallas_call(
        body,
        grid=(N // BN,),
        in_specs=[
            pl.BlockSpec((2, BN, d_in_h), lambda i: (0, i, 0)),
            pl.BlockSpec((2, BN, d_in_h), lambda i: (0, i, 0)),
            pl.BlockSpec((BN, 1), lambda i: (i, 0)),
            pl.BlockSpec((2, 1, d_in_h), lambda i: (0, 0, 0)),
            pl.BlockSpec((2 * d_in_h, d_out), lambda i: (0, 0)),
        ],
        out_specs=out_spec,
        out_shape=out_shape,
    )(agg, ys, dis2, b2d, w)


def _tc_final(aggp, ys, dis2, b2d):
    def body(a_ref, ys_ref, dis_ref, b_ref, out_ref):
        dis = dis_ref[...]
        a = a_ref[0] + a_ref[1]
        out_ref[...] = jnp.maximum((a + ys_ref[...]) * dis + b_ref[...], 0.0)

    return pl.pallas_call(
        body,
        grid=(N // BN,),
        in_specs=[
            pl.BlockSpec((2, BN, 128), lambda i: (0, i, 0)),
            pl.BlockSpec((BN, 128), lambda i: (i, 0)),
            pl.BlockSpec((BN, 1), lambda i: (i, 0)),
            pl.BlockSpec((1, 128), lambda i: (0, 0)),
        ],
        out_specs=pl.BlockSpec((BN, 128), lambda i: (i, 0)),
        out_shape=jax.ShapeDtypeStruct((N, 128), jnp.float32),
    )(aggp, ys, dis2, b2d)


def kernel(x, edge_index, edge_features, W1, b1, Wh, bh, W2, b2):
    src = edge_index[0].astype(jnp.int32)
    dst = edge_index[1].astype(jnp.int32)
    ew = edge_features.astype(jnp.float32)

    pad = E_PAD - E
    src_p = jnp.concatenate([src, jnp.zeros((pad,), jnp.int32)])
    dst_p = jnp.concatenate([dst, jnp.zeros((pad,), jnp.int32)])
    ew_p = jnp.concatenate([ew, jnp.zeros((pad,), jnp.float32)])
    src2 = jnp.stack([src_p, src_p + N]).reshape(2, ROWS_TOTAL, EDGE_COLS)
    dstr = dst_p.reshape(ROWS_TOTAL, EDGE_COLS)
    ewr = ew_p.reshape(ROWS_TOTAL, EDGE_COLS)

    degp = _sc_deg()(dstr, ewr).reshape(2, N, 128)

    b1_2d = b1.reshape(2, 1, 128)
    bh_2d = bh.reshape(2, 1, 128)
    b2_2d = b2.reshape(1, 128)

    ys1, dis2 = _tc_first(x, W1, degp)
    a1 = _sc_pass(True)(ys1.reshape(2 * N, 128), src2, dstr, ewr)
    ys2 = _tc_mid(a1.reshape(2, N, 128), ys1, dis2, b1_2d, Wh, 128, 256,
                  split_out=True)
    a2 = _sc_pass(True)(ys2.reshape(2 * N, 128), src2, dstr, ewr)
    ys3 = _tc_mid(a2.reshape(2, N, 128), ys2, dis2, bh_2d, W2, 128, 128,
                  split_out=False)
    a3p = _sc_pass(False)(ys3, src2[0], dstr, ewr).reshape(2, N, 128)
    return _tc_final(a3p, ys3, dis2, b2_2d)


# speedup vs baseline: 1.3876x; 1.1860x over previous
import functools

import jax
import jax.numpy as jnp
from jax import lax
from jax.experimental import pallas as pl
from jax.experimental.pallas import tpu as pltpu
from jax.experimental.pallas import tpu_sc as plsc

N = 10000
E = 320000
LANES = 16
EDGE_COLS = 128
TILES = 16
ROWS_PER_TILE = 160
ROWS_TOTAL = TILES * ROWS_PER_TILE
E_PAD = ROWS_TOTAL * EDGE_COLS
RPT_FULL = ROWS_TOTAL // 32
NCHUNK = 624
ZCHUNK = 16
CR = 16
CRH = CR // 2
DH = 128
GS = 4

BN = 1000


@functools.cache
def _sc_pass(feature_split):
    mesh = plsc.VectorSubcoreMesh(core_axis_name="c", subcore_axis_name="s")
    rpt = ROWS_PER_TILE if feature_split else RPT_FULL

    @functools.partial(
        pl.kernel,
        mesh=mesh,
        out_type=jax.ShapeDtypeStruct((2 * N, DH), jnp.float32),
        compiler_params=pltpu.CompilerParams(
            use_tc_tiling_on_sc=False, needs_layout_passes=False),
        scratch_types=[
            pltpu.VMEM((CR, EDGE_COLS), jnp.int32),
            pltpu.VMEM((CR, EDGE_COLS), jnp.int32),
            pltpu.VMEM((CR * EDGE_COLS,), jnp.float32),
            pltpu.VMEM((2, EDGE_COLS, DH // 2), jnp.uint32),
            pltpu.VMEM((EDGE_COLS, DH), jnp.float32),
            pltpu.VMEM_SHARED((N, DH), jnp.float32),
            pltpu.SemaphoreType.DMA,
            pltpu.SemaphoreType.DMA,
        ],
    )
    def agg(ys_hbm, src_hbm, dst_hbm, ew_hbm, zeros_hbm, out_hbm,
            src_v, dst_v, ew_v, rows_p, rows_f, acc_sh,
            sem0, sem1):
        c = lax.axis_index("c")
        s = lax.axis_index("s")

        n0 = s * NCHUNK
        last = N - (TILES - 1) * NCHUNK

        @pl.when(s < TILES - 1)
        def _():
            pltpu.sync_copy(zeros_hbm.at[pl.ds(n0, NCHUNK)],
                            acc_sh.at[pl.ds(n0, NCHUNK)])

        @pl.when(s == TILES - 1)
        def _():
            pltpu.sync_copy(zeros_hbm.at[pl.ds(n0, last)],
                            acc_sh.at[pl.ds(n0, last)])

        plsc.subcore_barrier()

        if feature_split:
            row0 = s * ROWS_PER_TILE
        else:
            row0 = (c * TILES + s) * RPT_FULL

        gsems = (sem0, sem1)

        def gather_start(b, j):
            pltpu.async_copy(ys_hbm.at[src_v.at[j]], rows_p.at[b], gsems[b])

        def gather_wait(b, j):
            pltpu.make_async_copy(ys_hbm.at[src_v.at[j]], rows_p.at[b],
                                  gsems[b]).wait()

        def process(b, j):
            for g in range(EDGE_COLS // LANES):
                wv = ew_v[pl.ds(j * EDGE_COLS + g * LANES, LANES)]
                e0 = g * LANES
                for i in range(LANES):
                    w = wv[i]
                    for k in range(DH // 32):
                        packed = rows_p[b, e0 + i, pl.ds(k * LANES, LANES)]
                        lo = plsc.bitcast(packed << 16, jnp.float32)
                        hi = plsc.bitcast(packed & jnp.uint32(0xFFFF0000),
                                          jnp.float32)
                        rows_f[e0 + i, pl.ds(k * 32, LANES)] = lo * w
                        rows_f[e0 + i, pl.ds(k * 32 + LANES, LANES)] = hi * w

            pltpu.sync_copy(rows_f, acc_sh.at[dst_v.at[j]], add=True)

        def chunk(ci, carry):
            r0 = row0 + ci * CR
            if feature_split:
                pltpu.sync_copy(src_hbm.at[c, pl.ds(r0, CR)], src_v)
            else:
                pltpu.sync_copy(src_hbm.at[pl.ds(r0, CR)], src_v)
            pltpu.sync_copy(dst_hbm.at[pl.ds(r0, CR)], dst_v)
            pltpu.sync_copy(ew_hbm.at[pl.ds(r0 * EDGE_COLS, CR * EDGE_COLS)],
                            ew_v)

            gather_start(0, 0)

            def pair(p, c2):
                j0 = p * 2
                gather_start(1, j0 + 1)
                gather_wait(0, j0)
                process(0, j0)

                @pl.when(p < CRH - 1)
                def _():
                    gather_start(0, j0 + 2)

                gather_wait(1, j0 + 1)
                process(1, j0 + 1)
                return c2

            lax.fori_loop(0, CRH, pair, 0)
            return carry

        lax.fori_loop(0, rpt // CR, chunk, 0)
        plsc.subcore_barrier()


        @pl.when(s < TILES - 1)
        def _():
            pltpu.sync_copy(acc_sh.at[pl.ds(n0, NCHUNK)],
                            out_hbm.at[pl.ds(c * N + n0, NCHUNK)])

        @pl.when(s == TILES - 1)
        def _():
            pltpu.sync_copy(acc_sh.at[pl.ds(n0, last)],
                            out_hbm.at[pl.ds(c * N + n0, last)])

    return agg


@functools.cache
def _sc_deg():
    mesh = plsc.VectorSubcoreMesh(core_axis_name="c", subcore_axis_name="s")

    @functools.partial(
        pl.kernel,
        mesh=mesh,
        out_type=jax.ShapeDtypeStruct((2 * N, DH), jnp.float32),
        scratch_types=[
            pltpu.VMEM((CR, EDGE_COLS), jnp.int32),
            pltpu.VMEM((CR, EDGE_COLS), jnp.float32),
            pltpu.VMEM((2, EDGE_COLS, DH), jnp.float32),
            pltpu.VMEM((ZCHUNK, DH), jnp.float32),
            pltpu.VMEM_SHARED((N, DH), jnp.float32),
            pltpu.SemaphoreType.DMA,
            pltpu.SemaphoreType.DMA,
        ],
    )
    def deg(dst_hbm, ew_hbm, out_hbm,
            dst_v, ew_v, rows_v, zero_v, acc_sh, tsem0, tsem1):
        c = lax.axis_index("c")
        s = lax.axis_index("s")

        zf = jnp.zeros((LANES,), jnp.float32)

        def zrow(r, carry):
            for k in range(DH // LANES):
                zero_v[r, pl.ds(k * LANES, LANES)] = zf
            return carry

        lax.fori_loop(0, ZCHUNK, zrow, 0)
        n0 = s * NCHUNK
        nz = jnp.where(s == TILES - 1, (N - (TILES - 1) * NCHUNK) // ZCHUNK,
                       NCHUNK // ZCHUNK)

        def zcopy(t, carry):
            pltpu.sync_copy(zero_v, acc_sh.at[pl.ds(n0 + t * ZCHUNK, ZCHUNK)])
            return carry

        lax.fori_loop(0, nz, zcopy, 0)
        plsc.subcore_barrier()

        row0 = (c * TILES + s) * RPT_FULL
        tsems = (tsem0, tsem1)

        def build(b, j):
            @plsc.parallel_loop(0, EDGE_COLS // LANES, unroll=2)
            def _(g):
                wv = ew_v[j, pl.ds(g * LANES, LANES)]
                e0 = g * LANES
                for i in range(LANES):
                    wb = jnp.full((LANES,), wv[i], jnp.float32)
                    for k in range(DH // LANES):
                        rows_v[b, e0 + i, pl.ds(k * LANES, LANES)] = wb

        def scatter_start(b, j):
            pltpu.async_copy(rows_v.at[b], acc_sh.at[dst_v.at[j]], tsems[b],
                             add=True)

        def scatter_wait(b):
            pltpu.make_async_copy(rows_v.at[b], acc_sh.at[dst_v.at[0]],
                                  tsems[b]).wait()

        def chunk(ci, carry):
            r0 = row0 + ci * CR
            pltpu.sync_copy(dst_hbm.at[pl.ds(r0, CR)], dst_v)
            pltpu.sync_copy(ew_hbm.at[pl.ds(r0, CR)], ew_v)

            def pair(p, c2):
                j0 = p * 2

                @pl.when(p > 0)
                def _():
                    scatter_wait(0)
                    scatter_wait(1)

                build(0, j0)
                scatter_start(0, j0)
                build(1, j0 + 1)
                scatter_start(1, j0 + 1)
                return c2

            lax.fori_loop(0, CRH, pair, 0)
            scatter_wait(0)
            scatter_wait(1)
            return carry

        lax.fori_loop(0, RPT_FULL // CR, chunk, 0)
        plsc.subcore_barrier()

        last = N - (TILES - 1) * NCHUNK

        @pl.when(s < TILES - 1)
        def _():
            pltpu.sync_copy(acc_sh.at[pl.ds(n0, NCHUNK)],
                            out_hbm.at[pl.ds(c * N + n0, NCHUNK)])

        @pl.when(s == TILES - 1)
        def _():
            pltpu.sync_copy(acc_sh.at[pl.ds(n0, last)],
                            out_hbm.at[pl.ds(c * N + n0, last)])

    return deg


def _tc_first(x, w1, degp):
    def body(x_ref, w_ref, deg_ref, ys_ref, dis_ref):
        deg = deg_ref[0, :, 0:1] + deg_ref[1, :, 0:1] + 1.0
        dis = lax.rsqrt(deg)
        xw = jnp.dot(x_ref[...], w_ref[...],
                     preferred_element_type=jnp.float32)
        ys = xw * dis
        ys_ref[0] = ys[:, :128]
        ys_ref[1] = ys[:, 128:]
        dis_ref[...] = dis

    return pl.pallas_call(
        body,
        grid=(N // BN,),
        in_specs=[
            pl.BlockSpec((BN, 128), lambda i: (i, 0)),
            pl.BlockSpec((128, 256), lambda i: (0, 0)),
            pl.BlockSpec((2, BN, 128), lambda i: (0, i, 0)),
        ],
        out_specs=[
            pl.BlockSpec((2, BN, 128), lambda i: (0, i, 0)),
            pl.BlockSpec((BN, 1), lambda i: (i, 0)),
        ],
        out_shape=[
            jax.ShapeDtypeStruct((2, N, 128), jnp.float32),
            jax.ShapeDtypeStruct((N, 1), jnp.float32),
        ],
    )(x, w1, degp)


def _tc_mid(agg, ys, dis2, b2d, w, d_in_h, d_out, split_out):
    doh = d_out // 2

    def body(a_ref, ys_ref, dis_ref, b_ref, w_ref, out_ref):
        dis = dis_ref[...]
        h0 = jnp.maximum((a_ref[0] + ys_ref[0]) * dis + b_ref[0], 0.0)
        h1 = jnp.maximum((a_ref[1] + ys_ref[1]) * dis + b_ref[1], 0.0)
        out = jnp.dot(h0, w_ref[:d_in_h, :],
                      preferred_element_type=jnp.float32)
        out = out + jnp.dot(h1, w_ref[d_in_h:, :],
                            preferred_element_type=jnp.float32)
        ysn = out * dis
        if split_out:
            out_ref[0] = ysn[:, :doh]
            out_ref[1] = ysn[:, doh:]
        else:
            out_ref[...] = ysn

    if split_out:
        out_spec = pl.BlockSpec((2, BN, doh), lambda i: (0, i, 0))
        out_shape = jax.ShapeDtypeStruct((2, N, doh), jnp.float32)
    else:
        out_spec = pl.BlockSpec((BN, d_out), lambda i: (i, 0))
        out_shape = jax.ShapeDtypeStruct((N, d_out), jnp.float32)

    return pl.pallas_call(
        body,
        grid=(N // BN,),
        in_specs=[
            pl.BlockSpec((2, BN, d_in_h), lambda i: (0, i, 0)),
            pl.BlockSpec((2, BN, d_in_h), lambda i: (0, i, 0)),
            pl.BlockSpec((BN, 1), lambda i: (i, 0)),
            pl.BlockSpec((2, 1, d_in_h), lambda i: (0, 0, 0)),
            pl.BlockSpec((2 * d_in_h, d_out), lambda i: (0, 0)),
        ],
        out_specs=out_spec,
        out_shape=out_shape,
    )(agg, ys, dis2, b2d, w)


def _tc_final(aggp, ys, dis2, b2d):
    def body(a_ref, ys_ref, dis_ref, b_ref, out_ref):
        dis = dis_ref[...]
        a = a_ref[0] + a_ref[1]
        out_ref[...] = jnp.maximum((a + ys_ref[...]) * dis + b_ref[...], 0.0)

    return pl.pallas_call(
        body,
        grid=(N // BN,),
        in_specs=[
            pl.BlockSpec((2, BN, 128), lambda i: (0, i, 0)),
            pl.BlockSpec((BN, 128), lambda i: (i, 0)),
            pl.BlockSpec((BN, 1), lambda i: (i, 0)),
            pl.BlockSpec((1, 128), lambda i: (0, 0)),
        ],
        out_specs=pl.BlockSpec((BN, 128), lambda i: (i, 0)),
        out_shape=jax.ShapeDtypeStruct((N, 128), jnp.float32),
    )(aggp, ys, dis2, b2d)


def kernel(x, edge_index, edge_features, W1, b1, Wh, bh, W2, b2):
    src = edge_index[0].astype(jnp.int32)
    dst = edge_index[1].astype(jnp.int32)
    ew = edge_features.astype(jnp.float32)

    pad = E_PAD - E
    src_p = jnp.concatenate([src, jnp.zeros((pad,), jnp.int32)])
    dst_p = jnp.concatenate([dst, jnp.zeros((pad,), jnp.int32)])
    ew_p = jnp.concatenate([ew, jnp.zeros((pad,), jnp.float32)])
    src2 = jnp.stack([src_p, src_p + N]).reshape(2, ROWS_TOTAL, EDGE_COLS)
    dstr = dst_p.reshape(ROWS_TOTAL, EDGE_COLS)
    ewr = ew_p.reshape(ROWS_TOTAL, EDGE_COLS)

    degp = _sc_deg()(dstr, ewr).reshape(2, N, 128)

    b1_2d = b1.reshape(2, 1, 128)
    bh_2d = bh.reshape(2, 1, 128)
    b2_2d = b2.reshape(1, 128)

    def pack_tab(t):
        tb = t.reshape(-1, 64, 2).astype(jnp.bfloat16)
        return jax.lax.bitcast_convert_type(tb, jnp.uint32)

    perm = jnp.array([32 * (m // 32) + (m % 32) // 2 + 16 * (m % 2)
                      for m in range(128)], jnp.int32)

    ys1, dis2 = _tc_first(x, W1, degp)
    zeros_n = jnp.zeros((N, DH), jnp.float32)
    a1 = _sc_pass(True)(pack_tab(ys1), src2, dstr, ew_p, zeros_n)
    a1 = jnp.take(a1, perm, axis=1)
    ys2 = _tc_mid(a1.reshape(2, N, 128), ys1, dis2, b1_2d, Wh, 128, 256,
                  split_out=True)
    a2 = _sc_pass(True)(pack_tab(ys2), src2, dstr, ew_p, zeros_n)
    a2 = jnp.take(a2, perm, axis=1)
    ys3 = _tc_mid(a2.reshape(2, N, 128), ys2, dis2, bh_2d, W2, 128, 128,
                  split_out=False)
    a3p = _sc_pass(False)(pack_tab(ys3), src2[0], dstr, ew_p, zeros_n)
    a3p = jnp.take(a3p, perm, axis=1).reshape(2, N, 128)
    return _tc_final(a3p, ys3, dis2, b2_2d)
